# baseline (speedup 1.0000x reference)
import jax
import jax.numpy as jnp
from jax import lax
from jax.experimental import pallas as pl
from jax.experimental.pallas import tpu as pltpu
from jax.experimental.pallas import tpu_sc as plsc

N = 100000
NC = 1
NS = 16
LANES = 16
CHUNK = 6272
HALF = CHUNK // 2
LAST = N - CHUNK
NSP = 10


def _sc_body(ae_hbm, idx_hbm, scale_hbm, shift_hbm, out_hbm,
             ae_v, idx_v, out_v, scale_v, shift_v, sem_a, sem_b, sem_o):
    wid = lax.axis_index("s") * NC + lax.axis_index("c")
    base = jnp.minimum(CHUNK * wid, LAST)
    i1 = pltpu.make_async_copy(idx_hbm.at[pl.ds(base, HALF)],
                               idx_v.at[pl.ds(0, HALF)], sem_a)
    a1 = pltpu.make_async_copy(ae_hbm.at[pl.ds(base, HALF)],
                               ae_v.at[pl.ds(0, HALF)], sem_a)
    t1 = pltpu.make_async_copy(scale_hbm, scale_v.at[pl.ds(0, NSP)], sem_a)
    t2 = pltpu.make_async_copy(shift_hbm, shift_v.at[pl.ds(0, NSP)], sem_a)
    i2 = pltpu.make_async_copy(idx_hbm.at[pl.ds(base + HALF, HALF)],
                               idx_v.at[pl.ds(HALF, HALF)], sem_b)
    a2 = pltpu.make_async_copy(ae_hbm.at[pl.ds(base + HALF, HALF)],
                               ae_v.at[pl.ds(HALF, HALF)], sem_b)
    i1.start(); a1.start(); t1.start(); t2.start(); i2.start(); a2.start()
    i1.wait(); a1.wait(); t1.wait(); t2.wait()

    def half(lo):
        @plsc.parallel_loop(lo, lo + HALF, step=LANES)
        def _(off):
            s = pl.ds(off, LANES)
            iv = idx_v[s]
            av = ae_v[s]
            sc = plsc.load_gather(scale_v, [iv])
            sh = plsc.load_gather(shift_v, [iv])
            out_v[s] = av * sc + sh

    half(0)
    o1 = pltpu.make_async_copy(out_v.at[pl.ds(0, HALF)],
                               out_hbm.at[pl.ds(base, HALF)], sem_o)
    o1.start()
    i2.wait(); a2.wait()
    half(HALF)
    o2 = pltpu.make_async_copy(out_v.at[pl.ds(HALF, HALF)],
                               out_hbm.at[pl.ds(base + HALF, HALF)], sem_o)
    o2.start()
    o1.wait(); o2.wait()


_sc_call = pl.kernel(
    _sc_body,
    out_type=jax.ShapeDtypeStruct((N,), jnp.float32),
    mesh=plsc.VectorSubcoreMesh(
        core_axis_name="c", subcore_axis_name="s",
        num_cores=NC, num_subcores=NS),
    compiler_params=pltpu.CompilerParams(needs_layout_passes=False),
    scratch_types=[
        pltpu.VMEM((CHUNK,), jnp.float32),
        pltpu.VMEM((CHUNK,), jnp.int32),
        pltpu.VMEM((CHUNK,), jnp.float32),
        pltpu.VMEM((LANES,), jnp.float32),
        pltpu.VMEM((LANES,), jnp.float32),
        pltpu.SemaphoreType.DMA,
        pltpu.SemaphoreType.DMA,
        pltpu.SemaphoreType.DMA,
    ],
)


def kernel(atomic_energy, atom_number, scale, shift):
    ae = atomic_energy.reshape(-1).astype(jnp.float32)
    idx = atom_number.reshape(-1).astype(jnp.int32)
    return _sc_call(ae, idx, scale.astype(jnp.float32),
                    shift.astype(jnp.float32))


# speedup vs baseline: 1.0427x; 1.0427x over previous
import jax
import jax.numpy as jnp
from jax import lax
from jax.experimental import pallas as pl
from jax.experimental.pallas import tpu as pltpu
from jax.experimental.pallas import tpu_sc as plsc

N = 100000
NC = 1
NS = 16
LANES = 16
CHUNK = 6272
HALF = CHUNK // 2
LAST = N - CHUNK
NSP = 10


def _sc_body(ae_hbm, idx_hbm, scale_hbm, shift_hbm, out_hbm,
             ae_v, idx_v, out_v, scale_v, shift_v, sem_a, sem_b, sem_o):
    wid = lax.axis_index("s") * NC + lax.axis_index("c")
    base = jnp.minimum(CHUNK * wid, LAST)
    i1 = pltpu.make_async_copy(idx_hbm.at[pl.ds(base, HALF)],
                               idx_v.at[pl.ds(0, HALF)], sem_a)
    a1 = pltpu.make_async_copy(ae_hbm.at[pl.ds(base, HALF)],
                               ae_v.at[pl.ds(0, HALF)], sem_a)
    t1 = pltpu.make_async_copy(scale_hbm, scale_v.at[pl.ds(0, NSP)], sem_a)
    t2 = pltpu.make_async_copy(shift_hbm, shift_v.at[pl.ds(0, NSP)], sem_a)
    i2 = pltpu.make_async_copy(idx_hbm.at[pl.ds(base + HALF, HALF)],
                               idx_v.at[pl.ds(HALF, HALF)], sem_b)
    a2 = pltpu.make_async_copy(ae_hbm.at[pl.ds(base + HALF, HALF)],
                               ae_v.at[pl.ds(HALF, HALF)], sem_b)
    i1.start(); a1.start(); t1.start(); t2.start(); i2.start(); a2.start()
    i1.wait(); a1.wait(); t1.wait(); t2.wait()

    def half(lo):
        @plsc.parallel_loop(lo, lo + HALF, step=LANES, unroll=2)
        def _(off):
            s = pl.ds(off, LANES)
            iv = idx_v[s]
            av = ae_v[s]
            sc = plsc.load_gather(scale_v, [iv])
            sh = plsc.load_gather(shift_v, [iv])
            out_v[s] = av * sc + sh

    half(0)
    o1 = pltpu.make_async_copy(out_v.at[pl.ds(0, HALF)],
                               out_hbm.at[pl.ds(base, HALF)], sem_o)
    o1.start()
    i2.wait(); a2.wait()
    half(HALF)
    o2 = pltpu.make_async_copy(out_v.at[pl.ds(HALF, HALF)],
                               out_hbm.at[pl.ds(base + HALF, HALF)], sem_o)
    o2.start()
    o1.wait(); o2.wait()


_sc_call = pl.kernel(
    _sc_body,
    out_type=jax.ShapeDtypeStruct((N,), jnp.float32),
    mesh=plsc.VectorSubcoreMesh(
        core_axis_name="c", subcore_axis_name="s",
        num_cores=NC, num_subcores=NS),
    compiler_params=pltpu.CompilerParams(needs_layout_passes=False),
    scratch_types=[
        pltpu.VMEM((CHUNK,), jnp.float32),
        pltpu.VMEM((CHUNK,), jnp.int32),
        pltpu.VMEM((CHUNK,), jnp.float32),
        pltpu.VMEM((LANES,), jnp.float32),
        pltpu.VMEM((LANES,), jnp.float32),
        pltpu.SemaphoreType.DMA,
        pltpu.SemaphoreType.DMA,
        pltpu.SemaphoreType.DMA,
    ],
)


def kernel(atomic_energy, atom_number, scale, shift):
    ae = atomic_energy.reshape(-1).astype(jnp.float32)
    idx = atom_number.reshape(-1).astype(jnp.int32)
    return _sc_call(ae, idx, scale.astype(jnp.float32),
                    shift.astype(jnp.float32))
